# baseline (speedup 1.0000x reference)
import functools

import jax
import jax.numpy as jnp
from jax import lax
from jax.experimental import pallas as pl
from jax.experimental.pallas import tpu as pltpu
from jax.experimental.pallas import tpu_sc as plsc

_WINDOW = 5
_NEG = 4

_NC = 2
_NS = 16
_NW = _NC * _NS


_CH = 640


def _sc_gather(table, idx_all):
    n = idx_all.shape[0]
    d = table.shape[1]
    per_w = n // _NW
    assert per_w % _CH == 0 and n % _NW == 0
    nch = per_w // _CH
    npair = (nch + 1) // 2

    mesh = plsc.VectorSubcoreMesh(
        core_axis_name="c", subcore_axis_name="s",
        num_cores=_NC, num_subcores=_NS)

    @functools.partial(
        pl.kernel,
        out_type=jax.ShapeDtypeStruct((n, d), jnp.float32),
        mesh=mesh,
        scratch_types=[
            pltpu.VMEM((per_w,), jnp.int32),
            pltpu.VMEM((2, _CH, d), jnp.float32),
            pltpu.SemaphoreType.DMA,
            pltpu.SemaphoreType.DMA,
        ],
        compiler_params=pltpu.CompilerParams(use_tc_tiling_on_sc=False),
    )
    def gather_kernel(table_hbm, idx_hbm, out_hbm, idx_v, rows_v, sem0, sem1):
        wid = lax.axis_index("s") * _NC + lax.axis_index("c")
        base = pl.multiple_of(wid * per_w, 8)
        pltpu.sync_copy(idx_hbm.at[pl.ds(base, per_w)], idx_v)
        sems = (sem0, sem1)

        def gather_chunk(i, b):
            off = pl.multiple_of(i * _CH, 8)
            return pltpu.make_async_copy(
                table_hbm.at[idx_v.at[pl.ds(off, _CH)]], rows_v.at[b], sems[b])

        gather_chunk(0, 0).start()

        def pair(j, carry):
            for b in range(2):
                i = 2 * j + b

                @pl.when(i + 1 < nch)
                def _():
                    gather_chunk(i + 1, 1 - b).start()

                @pl.when(i < nch)
                def _():
                    gather_chunk(i, b).wait()
                    pltpu.sync_copy(
                        rows_v.at[b],
                        out_hbm.at[pl.ds(pl.multiple_of(base + i * _CH, 8),
                                         _CH)])
            return carry

        lax.fori_loop(0, npair, pair, 0)

    return gather_kernel(table, idx_all)


def _transpose_body(in_ref, out_ref):
    out_ref[...] = in_ref[...].T


def _tc_row_major(table):
    n, d = table.shape
    nblk = 2048
    return pl.pallas_call(
        _transpose_body,
        grid=((n + nblk - 1) // nblk,),
        in_specs=[pl.BlockSpec((d, nblk), lambda i: (0, i))],
        out_specs=pl.BlockSpec((nblk, d), lambda i: (i, 0)),
        out_shape=jax.ShapeDtypeStruct((n, d), jnp.float32),
    )(table.T)


def _rowsum(prod, ones_row):
    return lax.dot_general(ones_row, prod, (((1,), (1,)), ((), ())),
                           preferred_element_type=jnp.float32)


def _softplus_masked_sum(x, mask):
    sp = jnp.maximum(x, 0.0) + jnp.log1p(jnp.exp(-jnp.abs(x)))
    return jnp.sum(jnp.where(mask, sp, 0.0))


def _loss_body(w_ref, n0_ref, n1_ref, n2_ref, n3_ref, out_ref, *,
               t, r, l, nb, scale):
    pid = pl.program_id(0)
    d = w_ref.shape[-1]
    rl = r * l
    w2 = w_ref[...]
    ones_row = jnp.ones((1, d), jnp.float32)
    acc = jnp.float32(0.0)
    for k in range(1, _WINDOW):
        prod = w2[:rl - k, :] * w2[k:, :]
        s = _rowsum(prod, ones_row)
        pos_t = lax.broadcasted_iota(jnp.int32, (1, rl - k), 1) % l
        acc += _softplus_masked_sum(-s, pos_t < t)
    for n_ref in (n0_ref, n1_ref, n2_ref, n3_ref):
        n2 = n_ref[...]
        nl = _rowsum(w2 * n2, ones_row)
        neg_t = lax.broadcasted_iota(jnp.int32, (1, rl), 1) % l
        acc += _softplus_masked_sum(nl, neg_t < t)

    @pl.when(pid == 0)
    def _():
        out_ref[...] = jnp.zeros_like(out_ref)

    out_ref[...] += acc.reshape(1, 1)

    @pl.when(pid == nb - 1)
    def _():
        out_ref[...] *= jnp.float32(scale)


def _tc_loss(gathered, b, l, t):
    d = gathered.shape[-1]
    r = 64
    nb = b // r
    n_terms = b * t * (_WINDOW - 1 + _NEG)
    body = functools.partial(_loss_body, t=t, r=r, l=l, nb=nb,
                             scale=1.0 / float(n_terms))

    def slab_spec(k):
        return pl.BlockSpec((r * l, d), lambda i, k=k: (k * nb + i, 0))

    out = pl.pallas_call(
        body,
        grid=(nb,),
        in_specs=[slab_spec(k) for k in range(1 + _NEG)],
        out_specs=pl.BlockSpec((1, 1), lambda i: (0, 0)),
        out_shape=jax.ShapeDtypeStruct((1, 1), jnp.float32),
    )(gathered, gathered, gathered, gathered, gathered)
    return out[0, 0]


def kernel(walk, table):
    b, l = walk.shape
    t = l - _WINDOW + 1
    bt = b * t
    n_nodes, d = table.shape
    neg = jax.random.randint(jax.random.key(42), (bt, _NEG), 1, n_nodes - 1,
                             dtype=jnp.int32)
    neg_pad = jnp.pad(neg.T.reshape(_NEG, b, t), ((0, 0), (0, 0), (0, l - t)))
    idx_all = jnp.concatenate([walk.reshape(-1), neg_pad.reshape(-1)])
    assert idx_all.shape[0] % (_NW * _CH) == 0
    table_rm = _tc_row_major(table)
    gathered = _sc_gather(table_rm, idx_all)
    return _tc_loss(gathered, b, l, t)


# speedup vs baseline: 1.9315x; 1.9315x over previous
import functools

import jax
import jax.numpy as jnp
from jax import lax
from jax.experimental import pallas as pl
from jax.experimental.pallas import tpu as pltpu
from jax.experimental.pallas import tpu_sc as plsc

_WINDOW = 5
_NEG = 4

_NC = 2
_NS = 16
_NW = _NC * _NS


_CH = 640


def _sc_gather(table, idx_all):
    n = idx_all.shape[0]
    d = table.shape[1]
    per_w = n // _NW
    assert per_w % _CH == 0 and n % _NW == 0
    nch = per_w // _CH
    npair = (nch + 1) // 2

    mesh = plsc.VectorSubcoreMesh(
        core_axis_name="c", subcore_axis_name="s",
        num_cores=_NC, num_subcores=_NS)

    @functools.partial(
        pl.kernel,
        out_type=jax.ShapeDtypeStruct((n, d), jnp.float32),
        mesh=mesh,
        scratch_types=[
            pltpu.VMEM((per_w,), jnp.int32),
            pltpu.VMEM((2, _CH, d), jnp.float32),
            pltpu.SemaphoreType.DMA,
            pltpu.SemaphoreType.DMA,
        ],
        compiler_params=pltpu.CompilerParams(use_tc_tiling_on_sc=False),
    )
    def gather_kernel(table_hbm, idx_hbm, out_hbm, idx_v, rows_v, sem0, sem1):
        wid = lax.axis_index("s") * _NC + lax.axis_index("c")
        base = pl.multiple_of(wid * per_w, 8)
        pltpu.sync_copy(idx_hbm.at[pl.ds(base, per_w)], idx_v)
        sems = (sem0, sem1)

        def gather_chunk(i, b):
            off = pl.multiple_of(i * _CH, 8)
            return pltpu.make_async_copy(
                table_hbm.at[idx_v.at[pl.ds(off, _CH)]], rows_v.at[b], sems[b])

        gather_chunk(0, 0).start()

        def pair(j, carry):
            for b in range(2):
                i = 2 * j + b

                @pl.when(i + 1 < nch)
                def _():
                    gather_chunk(i + 1, 1 - b).start()

                @pl.when(i < nch)
                def _():
                    gather_chunk(i, b).wait()
                    pltpu.sync_copy(
                        rows_v.at[b],
                        out_hbm.at[pl.ds(pl.multiple_of(base + i * _CH, 8),
                                         _CH)])
            return carry

        lax.fori_loop(0, npair, pair, 0)

    return gather_kernel(table, idx_all)


def _transpose_body(in_ref, out_ref):
    out_ref[...] = in_ref[...].T


def _tc_row_major(table):
    n, d = table.shape
    nblk = 2048
    return pl.pallas_call(
        _transpose_body,
        grid=((n + nblk - 1) // nblk,),
        in_specs=[pl.BlockSpec((d, nblk), lambda i: (0, i))],
        out_specs=pl.BlockSpec((nblk, d), lambda i: (i, 0)),
        out_shape=jax.ShapeDtypeStruct((n, d), jnp.float32),
    )(table.T)


def _rowsum(prod, ones_row):
    return lax.dot_general(ones_row, prod, (((1,), (1,)), ((), ())),
                           preferred_element_type=jnp.float32)


def _softplus_masked_sum(x, mask):
    sp = jnp.maximum(x, 0.0) + jnp.log1p(jnp.exp(-jnp.abs(x)))
    return jnp.sum(jnp.where(mask, sp, 0.0))


def _loss_body(w_ref, n0_ref, n1_ref, n2_ref, n3_ref, out_ref, *,
               t, r, l, nb, scale):
    pid = pl.program_id(0)
    d = w_ref.shape[-1]
    rl = r * l
    w2 = w_ref[...]
    ones_row = jnp.ones((1, d), jnp.float32)
    acc = jnp.float32(0.0)
    for k in range(1, _WINDOW):
        prod = w2[:rl - k, :] * w2[k:, :]
        s = _rowsum(prod, ones_row)
        pos_t = lax.broadcasted_iota(jnp.int32, (1, rl - k), 1) % l
        acc += _softplus_masked_sum(-s, pos_t < t)
    for n_ref in (n0_ref, n1_ref, n2_ref, n3_ref):
        n2 = n_ref[...]
        nl = _rowsum(w2 * n2, ones_row)
        neg_t = lax.broadcasted_iota(jnp.int32, (1, rl), 1) % l
        acc += _softplus_masked_sum(nl, neg_t < t)

    @pl.when(pid == 0)
    def _():
        out_ref[...] = jnp.zeros_like(out_ref)

    out_ref[...] += acc.reshape(1, 1)

    @pl.when(pid == nb - 1)
    def _():
        out_ref[...] *= jnp.float32(scale)


def _tc_loss(gathered, b, l, t):
    d = gathered.shape[-1]
    r = 64
    nb = b // r
    n_terms = b * t * (_WINDOW - 1 + _NEG)
    body = functools.partial(_loss_body, t=t, r=r, l=l, nb=nb,
                             scale=1.0 / float(n_terms))

    def slab_spec(k):
        return pl.BlockSpec((r * l, d), lambda i, k=k: (k * nb + i, 0))

    out = pl.pallas_call(
        body,
        grid=(nb,),
        in_specs=[slab_spec(k) for k in range(1 + _NEG)],
        out_specs=pl.BlockSpec((1, 1), lambda i: (0, 0)),
        out_shape=jax.ShapeDtypeStruct((1, 1), jnp.float32),
    )(gathered, gathered, gathered, gathered, gathered)
    return out[0, 0]


def kernel(walk, table):
    b, l = walk.shape
    t = l - _WINDOW + 1
    bt = b * t
    n_nodes, d = table.shape
    neg = jax.random.randint(jax.random.key(42), (bt, _NEG), 1, n_nodes - 1,
                             dtype=jnp.int32)
    pad_idx = jnp.broadcast_to(
        (jnp.arange(b, dtype=jnp.int32) * 719) % n_nodes, (_NEG, l - t, b))
    neg_pad = jnp.concatenate(
        [neg.T.reshape(_NEG, b, t),
         jnp.transpose(pad_idx, (0, 2, 1))], axis=2)
    idx_all = jnp.concatenate([walk.reshape(-1), neg_pad.reshape(-1)])
    assert idx_all.shape[0] % (_NW * _CH) == 0
    table_rm = _tc_row_major(table)
    gathered = _sc_gather(table_rm, idx_all)
    return _tc_loss(gathered, b, l, t)


# speedup vs baseline: 2.1895x; 1.1336x over previous
import functools

import jax
import jax.numpy as jnp
from jax import lax
from jax.experimental import pallas as pl
from jax.experimental.pallas import tpu as pltpu
from jax.experimental.pallas import tpu_sc as plsc

_WINDOW = 5
_NEG = 4

_NC = 2
_NS = 16
_NW = _NC * _NS


_CH = 640


def _sc_gather(table, idx_all):
    n = idx_all.shape[0]
    d = table.shape[1]
    per_w = n // _NW
    assert per_w % _CH == 0 and n % _NW == 0
    nch = per_w // _CH
    npair = (nch + 1) // 2

    mesh = plsc.VectorSubcoreMesh(
        core_axis_name="c", subcore_axis_name="s",
        num_cores=_NC, num_subcores=_NS)

    @functools.partial(
        pl.kernel,
        out_type=jax.ShapeDtypeStruct((n, d), jnp.float32),
        mesh=mesh,
        scratch_types=[
            pltpu.VMEM((per_w,), jnp.int32),
            pltpu.VMEM((2, _CH, d), jnp.float32),
            pltpu.SemaphoreType.DMA,
            pltpu.SemaphoreType.DMA,
        ],
        compiler_params=pltpu.CompilerParams(use_tc_tiling_on_sc=False),
    )
    def gather_kernel(table_hbm, idx_hbm, out_hbm, idx_v, rows_v, sem0, sem1):
        wid = lax.axis_index("s") * _NC + lax.axis_index("c")
        base = pl.multiple_of(wid * per_w, 8)
        pltpu.sync_copy(idx_hbm.at[pl.ds(base, per_w)], idx_v)
        sems = (sem0, sem1)

        def gather_chunk(i, b):
            off = pl.multiple_of(i * _CH, 8)
            return pltpu.make_async_copy(
                table_hbm.at[idx_v.at[pl.ds(off, _CH)]], rows_v.at[b], sems[b])

        gather_chunk(0, 0).start()

        def pair(j, carry):
            for b in range(2):
                i = 2 * j + b

                @pl.when(i + 1 < nch)
                def _():
                    gather_chunk(i + 1, 1 - b).start()

                @pl.when(i < nch)
                def _():
                    gather_chunk(i, b).wait()
                    pltpu.sync_copy(
                        rows_v.at[b],
                        out_hbm.at[pl.ds(pl.multiple_of(base + i * _CH, 8),
                                         _CH)])
            return carry

        lax.fori_loop(0, npair, pair, 0)

    return gather_kernel(table, idx_all)


def _transpose_body(in_ref, out_ref):
    out_ref[...] = in_ref[...].T


def _tc_row_major(table):
    n, d = table.shape
    nblk = 8192
    return pl.pallas_call(
        _transpose_body,
        grid=((n + nblk - 1) // nblk,),
        in_specs=[pl.BlockSpec((d, nblk), lambda i: (0, i))],
        out_specs=pl.BlockSpec((nblk, d), lambda i: (i, 0)),
        out_shape=jax.ShapeDtypeStruct((n, d), jnp.float32),
    )(table.T)


def _rowsum(prod, ones_row):
    return lax.dot_general(ones_row, prod, (((1,), (1,)), ((), ())),
                           preferred_element_type=jnp.float32)


def _softplus_masked_sum(x, mask):
    sp = jnp.maximum(x, 0.0) + jnp.log1p(jnp.exp(-jnp.abs(x)))
    return jnp.sum(jnp.where(mask, sp, 0.0))


def _loss_body(w_ref, n0_ref, n1_ref, n2_ref, n3_ref, out_ref, *,
               t, r, l, nb, scale):
    pid = pl.program_id(0)
    d = w_ref.shape[-1]
    rl = r * l
    w2 = w_ref[...]
    ones_row = jnp.ones((1, d), jnp.float32)
    acc = jnp.float32(0.0)
    for k in range(1, _WINDOW):
        prod = w2[:rl - k, :] * w2[k:, :]
        s = _rowsum(prod, ones_row)
        pos_t = lax.broadcasted_iota(jnp.int32, (1, rl - k), 1) % l
        acc += _softplus_masked_sum(-s, pos_t < t)
    for n_ref in (n0_ref, n1_ref, n2_ref, n3_ref):
        n2 = n_ref[...]
        nl = _rowsum(w2 * n2, ones_row)
        neg_t = lax.broadcasted_iota(jnp.int32, (1, rl), 1) % l
        acc += _softplus_masked_sum(nl, neg_t < t)

    @pl.when(pid == 0)
    def _():
        out_ref[...] = jnp.zeros_like(out_ref)

    out_ref[...] += acc.reshape(1, 1)

    @pl.when(pid == nb - 1)
    def _():
        out_ref[...] *= jnp.float32(scale)


def _tc_loss(gathered, b, l, t):
    d = gathered.shape[-1]
    r = 64
    nb = b // r
    n_terms = b * t * (_WINDOW - 1 + _NEG)
    body = functools.partial(_loss_body, t=t, r=r, l=l, nb=nb,
                             scale=1.0 / float(n_terms))

    def slab_spec(k):
        return pl.BlockSpec((r * l, d), lambda i, k=k: (k * nb + i, 0))

    out = pl.pallas_call(
        body,
        grid=(nb,),
        in_specs=[slab_spec(k) for k in range(1 + _NEG)],
        out_specs=pl.BlockSpec((1, 1), lambda i: (0, 0)),
        out_shape=jax.ShapeDtypeStruct((1, 1), jnp.float32),
    )(gathered, gathered, gathered, gathered, gathered)
    return out[0, 0]


def kernel(walk, table):
    b, l = walk.shape
    t = l - _WINDOW + 1
    bt = b * t
    n_nodes, d = table.shape
    neg = jax.random.randint(jax.random.key(42), (bt, _NEG), 1, n_nodes - 1,
                             dtype=jnp.int32)
    pad_idx = jnp.broadcast_to(
        (jnp.arange(b, dtype=jnp.int32) * 719) % n_nodes, (_NEG, l - t, b))
    neg_pad = jnp.concatenate(
        [neg.T.reshape(_NEG, b, t),
         jnp.transpose(pad_idx, (0, 2, 1))], axis=2)
    idx_all = jnp.concatenate([walk.reshape(-1), neg_pad.reshape(-1)])
    assert idx_all.shape[0] % (_NW * _CH) == 0
    table_rm = _tc_row_major(table)
    gathered = _sc_gather(table_rm, idx_all)
    return _tc_loss(gathered, b, l, t)


# speedup vs baseline: 2.4038x; 1.0979x over previous
import functools

import jax
import jax.numpy as jnp
from jax import lax
from jax.experimental import pallas as pl
from jax.experimental.pallas import tpu as pltpu
from jax.experimental.pallas import tpu_sc as plsc

_WINDOW = 5
_NEG = 4

_NC = 2
_NS = 16
_NW = _NC * _NS


_CH = 640


def _sc_gather(table, idx_all):
    n = idx_all.shape[0]
    d = table.shape[1]
    per_w = n // _NW
    assert per_w % _CH == 0 and n % _NW == 0
    nch = per_w // _CH
    npair = (nch + 1) // 2

    mesh = plsc.VectorSubcoreMesh(
        core_axis_name="c", subcore_axis_name="s",
        num_cores=_NC, num_subcores=_NS)

    @functools.partial(
        pl.kernel,
        out_type=jax.ShapeDtypeStruct((n, d), jnp.uint32),
        mesh=mesh,
        scratch_types=[
            pltpu.VMEM((per_w,), jnp.int32),
            pltpu.VMEM((2, _CH, d), jnp.uint32),
            pltpu.SemaphoreType.DMA,
            pltpu.SemaphoreType.DMA,
        ],
        compiler_params=pltpu.CompilerParams(use_tc_tiling_on_sc=False),
    )
    def gather_kernel(table_hbm, idx_hbm, out_hbm, idx_v, rows_v, sem0, sem1):
        wid = lax.axis_index("s") * _NC + lax.axis_index("c")
        base = pl.multiple_of(wid * per_w, 8)
        pltpu.sync_copy(idx_hbm.at[pl.ds(base, per_w)], idx_v)
        sems = (sem0, sem1)

        def gather_chunk(i, b):
            off = pl.multiple_of(i * _CH, 8)
            return pltpu.make_async_copy(
                table_hbm.at[idx_v.at[pl.ds(off, _CH)]], rows_v.at[b], sems[b])

        gather_chunk(0, 0).start()

        def pair(j, carry):
            for b in range(2):
                i = 2 * j + b

                @pl.when(i + 1 < nch)
                def _():
                    gather_chunk(i + 1, 1 - b).start()

                @pl.when(i < nch)
                def _():
                    gather_chunk(i, b).wait()
                    pltpu.sync_copy(
                        rows_v.at[b],
                        out_hbm.at[pl.ds(pl.multiple_of(base + i * _CH, 8),
                                         _CH)])
            return carry

        lax.fori_loop(0, npair, pair, 0)

    return gather_kernel(table, idx_all)


def _transpose_body(in_ref, out_ref):
    x = in_ref[...]
    d = x.shape[0]
    b = lax.bitcast_convert_type(x, jnp.uint32)
    rnd = (b + jnp.uint32(0x7FFF) + ((b >> 16) & jnp.uint32(1))) >> 16
    u = (rnd[d // 2:, :] << 16) | rnd[:d // 2, :]
    out_ref[...] = u.T


def _tc_row_major(table):
    n, d = table.shape
    nblk = 8192
    return pl.pallas_call(
        _transpose_body,
        grid=((n + nblk - 1) // nblk,),
        in_specs=[pl.BlockSpec((d, nblk), lambda i: (0, i))],
        out_specs=pl.BlockSpec((nblk, d // 2), lambda i: (i, 0)),
        out_shape=jax.ShapeDtypeStruct((n, d // 2), jnp.uint32),
    )(table.T)


def _rowsum(prod, ones_row):
    return lax.dot_general(ones_row, prod, (((1,), (1,)), ((), ())),
                           preferred_element_type=jnp.float32)


def _softplus_masked_sum(x, mask):
    sp = jnp.maximum(x, 0.0) + jnp.log1p(jnp.exp(-jnp.abs(x)))
    return jnp.sum(jnp.where(mask, sp, 0.0))


def _unpack2_f32(u):
    lo = lax.bitcast_convert_type(u << 16, jnp.float32)
    hi = lax.bitcast_convert_type(u & jnp.uint32(0xFFFF0000), jnp.float32)
    return lo, hi


def _loss_body(w_ref, n0_ref, n1_ref, n2_ref, n3_ref, out_ref, *,
               t, r, l, nb, scale):
    pid = pl.program_id(0)
    rl = r * l
    dh = w_ref.shape[-1]
    wlo, whi = _unpack2_f32(w_ref[...])
    ones_row = jnp.ones((1, dh), jnp.float32)
    acc = jnp.float32(0.0)
    for k in range(1, _WINDOW):
        prod = wlo[:rl - k, :] * wlo[k:, :] + whi[:rl - k, :] * whi[k:, :]
        s = _rowsum(prod, ones_row)
        pos_t = lax.broadcasted_iota(jnp.int32, (1, rl - k), 1) % l
        acc += _softplus_masked_sum(-s, pos_t < t)
    for n_ref in (n0_ref, n1_ref, n2_ref, n3_ref):
        nlo, nhi = _unpack2_f32(n_ref[...])
        nl = _rowsum(wlo * nlo + whi * nhi, ones_row)
        neg_t = lax.broadcasted_iota(jnp.int32, (1, rl), 1) % l
        acc += _softplus_masked_sum(nl, neg_t < t)

    @pl.when(pid == 0)
    def _():
        out_ref[...] = jnp.zeros_like(out_ref)

    out_ref[...] += acc.reshape(1, 1)

    @pl.when(pid == nb - 1)
    def _():
        out_ref[...] *= jnp.float32(scale)


def _tc_loss(gathered, b, l, t):
    d = gathered.shape[-1]
    r = 64
    nb = b // r
    n_terms = b * t * (_WINDOW - 1 + _NEG)
    body = functools.partial(_loss_body, t=t, r=r, l=l, nb=nb,
                             scale=1.0 / float(n_terms))

    def slab_spec(k):
        return pl.BlockSpec((r * l, d), lambda i, k=k: (k * nb + i, 0))

    out = pl.pallas_call(
        body,
        grid=(nb,),
        in_specs=[slab_spec(k) for k in range(1 + _NEG)],
        out_specs=pl.BlockSpec((1, 1), lambda i: (0, 0)),
        out_shape=jax.ShapeDtypeStruct((1, 1), jnp.float32),
    )(gathered, gathered, gathered, gathered, gathered)
    return out[0, 0]


def kernel(walk, table):
    b, l = walk.shape
    t = l - _WINDOW + 1
    bt = b * t
    n_nodes, d = table.shape
    neg = jax.random.randint(jax.random.key(42), (bt, _NEG), 1, n_nodes - 1,
                             dtype=jnp.int32)
    pad_idx = jnp.broadcast_to(
        (jnp.arange(b, dtype=jnp.int32) * 719) % n_nodes, (_NEG, l - t, b))
    neg_pad = jnp.concatenate(
        [neg.T.reshape(_NEG, b, t),
         jnp.transpose(pad_idx, (0, 2, 1))], axis=2)
    idx_all = jnp.concatenate([walk.reshape(-1), neg_pad.reshape(-1)])
    assert idx_all.shape[0] % (_NW * _CH) == 0
    table_rm = _tc_row_major(table)
    gathered = _sc_gather(table_rm, idx_all)
    return _tc_loss(gathered, b, l, t)


# speedup vs baseline: 2.4367x; 1.0137x over previous
import functools

import jax
import jax.numpy as jnp
from jax import lax
from jax.experimental import pallas as pl
from jax.experimental.pallas import tpu as pltpu
from jax.experimental.pallas import tpu_sc as plsc

_WINDOW = 5
_NEG = 4

_NC = 2
_NS = 16
_NW = _NC * _NS


_CH = 640


def _sc_gather(table, idx_all):
    n = idx_all.shape[0]
    d = table.shape[1]
    per_w = n // _NW
    assert per_w % _CH == 0 and n % _NW == 0
    nch = per_w // _CH
    npair = (nch + 1) // 2

    mesh = plsc.VectorSubcoreMesh(
        core_axis_name="c", subcore_axis_name="s",
        num_cores=_NC, num_subcores=_NS)

    @functools.partial(
        pl.kernel,
        out_type=jax.ShapeDtypeStruct((n, d), jnp.uint32),
        mesh=mesh,
        scratch_types=[
            pltpu.VMEM((per_w,), jnp.int32),
            pltpu.VMEM((2, _CH, d), jnp.uint32),
            pltpu.SemaphoreType.DMA,
            pltpu.SemaphoreType.DMA,
        ],
        compiler_params=pltpu.CompilerParams(use_tc_tiling_on_sc=False),
    )
    def gather_kernel(table_hbm, idx_hbm, out_hbm, idx_v, rows_v, sem0, sem1):
        wid = lax.axis_index("s") * _NC + lax.axis_index("c")
        base = pl.multiple_of(wid * per_w, 8)
        pltpu.sync_copy(idx_hbm.at[pl.ds(base, per_w)], idx_v)
        sems = (sem0, sem1)

        def gather_chunk(i, b):
            off = pl.multiple_of(i * _CH, 8)
            return pltpu.make_async_copy(
                table_hbm.at[idx_v.at[pl.ds(off, _CH)]], rows_v.at[b], sems[b])

        gather_chunk(0, 0).start()

        def pair(j, carry):
            for b in range(2):
                i = 2 * j + b

                @pl.when(i + 1 < nch)
                def _():
                    gather_chunk(i + 1, 1 - b).start()

                @pl.when(i < nch)
                def _():
                    gather_chunk(i, b).wait()
                    pltpu.sync_copy(
                        rows_v.at[b],
                        out_hbm.at[pl.ds(pl.multiple_of(base + i * _CH, 8),
                                         _CH)])
            return carry

        lax.fori_loop(0, npair, pair, 0)

    return gather_kernel(table, idx_all)


def _transpose_body(in_ref, out_ref):
    x = in_ref[...]
    d = x.shape[0]
    b = lax.bitcast_convert_type(x, jnp.uint32)
    rnd = (b + jnp.uint32(0x7FFF) + ((b >> 16) & jnp.uint32(1))) >> 16
    u = (rnd[d // 2:, :] << 16) | rnd[:d // 2, :]
    out_ref[...] = u.T


def _tc_row_major(table):
    n, d = table.shape
    nblk = 16384
    return pl.pallas_call(
        _transpose_body,
        grid=((n + nblk - 1) // nblk,),
        in_specs=[pl.BlockSpec((d, nblk), lambda i: (0, i))],
        out_specs=pl.BlockSpec((nblk, d // 2), lambda i: (i, 0)),
        out_shape=jax.ShapeDtypeStruct((n, d // 2), jnp.uint32),
    )(table.T)


def _rowsum(prod, ones_row):
    return lax.dot_general(ones_row, prod, (((1,), (1,)), ((), ())),
                           preferred_element_type=jnp.float32)


def _softplus_masked_sum(x, mask):
    sp = jnp.maximum(x, 0.0) + jnp.log1p(jnp.exp(-jnp.abs(x)))
    return jnp.sum(jnp.where(mask, sp, 0.0))


def _unpack2_f32(u):
    lo = lax.bitcast_convert_type(u << 16, jnp.float32)
    hi = lax.bitcast_convert_type(u & jnp.uint32(0xFFFF0000), jnp.float32)
    return lo, hi


def _loss_body(w_ref, n0_ref, n1_ref, n2_ref, n3_ref, out_ref, *,
               t, r, l, nb, scale):
    pid = pl.program_id(0)
    rl = r * l
    dh = w_ref.shape[-1]
    wlo, whi = _unpack2_f32(w_ref[...])
    ones_row = jnp.ones((1, dh), jnp.float32)
    acc = jnp.float32(0.0)
    for k in range(1, _WINDOW):
        prod = wlo[:rl - k, :] * wlo[k:, :] + whi[:rl - k, :] * whi[k:, :]
        s = _rowsum(prod, ones_row)
        pos_t = lax.broadcasted_iota(jnp.int32, (1, rl - k), 1) % l
        acc += _softplus_masked_sum(-s, pos_t < t)
    for n_ref in (n0_ref, n1_ref, n2_ref, n3_ref):
        nlo, nhi = _unpack2_f32(n_ref[...])
        nl = _rowsum(wlo * nlo + whi * nhi, ones_row)
        neg_t = lax.broadcasted_iota(jnp.int32, (1, rl), 1) % l
        acc += _softplus_masked_sum(nl, neg_t < t)

    @pl.when(pid == 0)
    def _():
        out_ref[...] = jnp.zeros_like(out_ref)

    out_ref[...] += acc.reshape(1, 1)

    @pl.when(pid == nb - 1)
    def _():
        out_ref[...] *= jnp.float32(scale)


def _tc_loss(gathered, b, l, t):
    d = gathered.shape[-1]
    r = 64
    nb = b // r
    n_terms = b * t * (_WINDOW - 1 + _NEG)
    body = functools.partial(_loss_body, t=t, r=r, l=l, nb=nb,
                             scale=1.0 / float(n_terms))

    def slab_spec(k):
        return pl.BlockSpec((r * l, d), lambda i, k=k: (k * nb + i, 0))

    out = pl.pallas_call(
        body,
        grid=(nb,),
        in_specs=[slab_spec(k) for k in range(1 + _NEG)],
        out_specs=pl.BlockSpec((1, 1), lambda i: (0, 0)),
        out_shape=jax.ShapeDtypeStruct((1, 1), jnp.float32),
    )(gathered, gathered, gathered, gathered, gathered)
    return out[0, 0]


def kernel(walk, table):
    b, l = walk.shape
    t = l - _WINDOW + 1
    bt = b * t
    n_nodes, d = table.shape
    neg = jax.random.randint(jax.random.key(42), (bt, _NEG), 1, n_nodes - 1,
                             dtype=jnp.int32)
    pad_idx = jnp.broadcast_to(
        (jnp.arange(b, dtype=jnp.int32) * 719) % n_nodes, (_NEG, l - t, b))
    neg_pad = jnp.concatenate(
        [neg.T.reshape(_NEG, b, t),
         jnp.transpose(pad_idx, (0, 2, 1))], axis=2)
    idx_all = jnp.concatenate([walk.reshape(-1), neg_pad.reshape(-1)])
    assert idx_all.shape[0] % (_NW * _CH) == 0
    table_rm = _tc_row_major(table)
    gathered = _sc_gather(table_rm, idx_all)
    return _tc_loss(gathered, b, l, t)
